# baseline (speedup 1.0000x reference)
import functools

import jax
import jax.numpy as jnp
from jax import lax
from jax.experimental import pallas as pl
from jax.experimental.pallas import tpu as pltpu
from jax.experimental.pallas import tpu_sc as plsc

N = 10000
G = 64
D = 128
NC = 2
NS = 16
NW = NC * NS
K = 128
NPAD = 10240
DUMMY = 10016
ROWS_PER_TILE = NPAD // NS
RB = 256
NBLK = NPAD // RB


@functools.lru_cache(maxsize=None)
def _make_prop(chunks0: int, chunks1: int):
  chunks = max(chunks0, chunks1)
  mesh = plsc.VectorSubcoreMesh(core_axis_name="c", subcore_axis_name="s")

  @functools.partial(
      pl.kernel,
      out_type=jax.ShapeDtypeStruct((NC, NPAD, D), jnp.float32),
      mesh=mesh,
      scratch_types=[
          pltpu.VMEM((chunks, K), jnp.int32),
          pltpu.VMEM((chunks, K), jnp.int32),
          pltpu.VMEM((K, D), jnp.float32),
          pltpu.VMEM_SHARED((NPAD, D), jnp.float32),
          pltpu.SemaphoreType.DMA,
          pltpu.SemaphoreType.DMA,
      ],
  )
  def prop(c_hbm, src_hbm, dst_hbm, zeros_hbm, out_hbm,
           src_v, dst_v, buf, acc, sem, sem_pre):
    cid = lax.axis_index("c")
    sid = lax.axis_index("s")
    wid = cid * NS + sid
    base = sid * ROWS_PER_TILE
    pltpu.async_copy(src_hbm.at[wid], src_v, sem_pre)
    pltpu.async_copy(dst_hbm.at[wid], dst_v, sem_pre)
    for z in range(ROWS_PER_TILE // 128):
      pltpu.async_copy(zeros_hbm, acc.at[pl.ds(base + z * 128, 128)], sem_pre)
    pltpu.make_async_copy(src_hbm.at[wid], src_v, sem_pre).wait()
    pltpu.make_async_copy(dst_hbm.at[wid], dst_v, sem_pre).wait()
    for z in range(ROWS_PER_TILE // 128):
      pltpu.make_async_copy(zeros_hbm,
                            acc.at[pl.ds(base + z * 128, 128)], sem_pre).wait()
    plsc.subcore_barrier()

    def body(j, carry):
      pltpu.async_copy(c_hbm.at[src_v.at[j]], buf, sem).wait()
      pltpu.sync_copy(buf, acc.at[dst_v.at[j]], add=True)
      return carry

    nloc = jnp.where(cid == 0, chunks0, chunks1)
    lax.fori_loop(0, nloc, body, 0)
    plsc.subcore_barrier()
    pltpu.sync_copy(acc.at[pl.ds(base, ROWS_PER_TILE)],
                    out_hbm.at[cid, pl.ds(base, ROWS_PER_TILE)])

  return prop


@functools.lru_cache(maxsize=None)
def _make_deg(chunks: int):
  mesh = plsc.VectorSubcoreMesh(core_axis_name="c", subcore_axis_name="s")

  @functools.partial(
      pl.kernel,
      out_type=jax.ShapeDtypeStruct((NW, NPAD // 16, 16), jnp.float32),
      mesh=mesh,
      scratch_types=[
          pltpu.VMEM((chunks * K,), jnp.int32),
          pltpu.VMEM((NPAD // 16, 16), jnp.float32),
      ],
      compiler_params=pltpu.CompilerParams(needs_layout_passes=False),
  )
  def deg(dst_hbm, zeros_hbm, out_hbm, dst_v, hist):
    cid = lax.axis_index("c")
    sid = lax.axis_index("s")
    wid = cid * NS + sid
    pltpu.sync_copy(dst_hbm.at[wid], dst_v)
    pltpu.sync_copy(zeros_hbm, hist)
    ones16 = jnp.ones((16,), jnp.float32)

    def body(i, carry):
      idx = dst_v[pl.ds(i * 16, 16)]
      plsc.addupdate_scatter(hist, [idx >> 4, idx & 15], ones16)
      return carry

    lax.fori_loop(0, (chunks * K) // 16, body, 0)
    pltpu.sync_copy(hist, out_hbm.at[wid])

  return deg


def _dinv_block(deg_blk):
  d = jnp.sum(deg_blk, axis=0)
  dinv16 = jnp.where(d > 0.0, lax.rsqrt(jnp.maximum(d, 1.0)), 0.0)
  sub = lax.broadcasted_iota(jnp.int32, (RB, 16), 0) // 16
  sel = (sub == lax.broadcasted_iota(jnp.int32, (RB, 16), 1)
         ).astype(jnp.float32)
  a = lax.dot_general(sel, dinv16, (((1,), (0,)), ((), ())),
                      preferred_element_type=jnp.float32)
  lane = lax.broadcasted_iota(jnp.int32, (RB, D), 0) % 16
  out = jnp.zeros((RB, D), jnp.float32)
  for b in range(16):
    out = out + jnp.where(lane == b, a[:, b:b + 1], 0.0)
  return out


def _scale_x_body(x_ref, deg_ref, c_ref, dinv_ref):
  dinv = _dinv_block(deg_ref[...])
  dinv_ref[...] = dinv
  c_ref[...] = x_ref[...] * dinv


def _scale_x(x, deg):
  return pl.pallas_call(
      _scale_x_body,
      grid=(NBLK,),
      in_specs=[
          pl.BlockSpec((RB, D), lambda i: (i, 0)),
          pl.BlockSpec((NW, RB // 16, 16), lambda i: (0, i, 0)),
      ],
      out_specs=[
          pl.BlockSpec((RB, D), lambda i: (i, 0)),
          pl.BlockSpec((RB, D), lambda i: (i, 0)),
      ],
      out_shape=[
          jax.ShapeDtypeStruct((NPAD, D), jnp.float32),
          jax.ShapeDtypeStruct((NPAD, D), jnp.float32),
      ],
      compiler_params=pltpu.CompilerParams(skip_device_barrier=True),
  )(x, deg)


def _layer_body(s_ref, dinv_ref, w_ref, b_ref, o_ref):
  dinv = dinv_ref[...]
  sp = (s_ref[0] + s_ref[1]) * dinv
  h = lax.dot_general(sp, w_ref[...], (((1,), (1,)), ((), ())),
                      preferred_element_type=jnp.float32)
  o_ref[...] = jnp.maximum(h + b_ref[...], 0.0) * dinv


def _layer(s, dinv_b, W, b):
  return pl.pallas_call(
      _layer_body,
      grid=(NBLK,),
      in_specs=[
          pl.BlockSpec((NC, RB, D), lambda i: (0, i, 0)),
          pl.BlockSpec((RB, D), lambda i: (i, 0)),
          pl.BlockSpec((D, D), lambda i: (0, 0)),
          pl.BlockSpec((1, D), lambda i: (0, 0)),
      ],
      out_specs=pl.BlockSpec((RB, D), lambda i: (i, 0)),
      out_shape=jax.ShapeDtypeStruct((NPAD, D), jnp.float32),
      compiler_params=pltpu.CompilerParams(skip_device_barrier=True),
  )(s, dinv_b, W, b)


def _pool_body(s_ref, dinv_ref, batch_ref, w_ref, b_ref, o_ref, s_acc, c_acc):
  i = pl.program_id(0)

  @pl.when(i == 0)
  def _():
    s_acc[...] = jnp.zeros_like(s_acc)
    c_acc[...] = jnp.zeros_like(c_acc)

  p = (s_ref[0] + s_ref[1]) * dinv_ref[...]
  bid = batch_ref[...]
  onehot = (lax.broadcasted_iota(jnp.int32, (RB, G), 1) == bid
            ).astype(jnp.float32)
  s_acc[...] += lax.dot_general(onehot, p, (((0,), (0,)), ((), ())),
                                preferred_element_type=jnp.float32)
  c_acc[...] += lax.dot_general(onehot, jnp.ones((RB, D), jnp.float32),
                                (((0,), (0,)), ((), ())),
                                preferred_element_type=jnp.float32)

  @pl.when(i == NBLK - 1)
  def _():
    cnt = c_acc[...]
    pooled = s_acc[...] / jnp.maximum(cnt, 1.0)
    res = lax.dot_general(pooled, w_ref[...], (((1,), (1,)), ((), ())),
                          preferred_element_type=jnp.float32)
    o_ref[...] = res + jnp.where(cnt > 0.0, b_ref[...], 0.0)


def _pool(s, dinv_b, batch_pad, W, b):
  return pl.pallas_call(
      _pool_body,
      grid=(NBLK,),
      in_specs=[
          pl.BlockSpec((NC, RB, D), lambda i: (0, i, 0)),
          pl.BlockSpec((RB, D), lambda i: (i, 0)),
          pl.BlockSpec((RB, 1), lambda i: (i, 0)),
          pl.BlockSpec((D, D), lambda i: (0, 0)),
          pl.BlockSpec((1, D), lambda i: (0, 0)),
      ],
      out_specs=pl.BlockSpec((G, D), lambda i: (0, 0)),
      out_shape=jax.ShapeDtypeStruct((G, D), jnp.float32),
      scratch_shapes=[
          pltpu.VMEM((G, D), jnp.float32),
          pltpu.VMEM((G, D), jnp.float32),
      ],
      compiler_params=pltpu.CompilerParams(skip_device_barrier=True),
  )(s, dinv_b, batch_pad, W, b)


def kernel(x, edge_index, batch, W1, b1, W2, b2, W3, b3):
  loops = jnp.arange(N, dtype=jnp.int32)
  src = jnp.concatenate([edge_index[0].astype(jnp.int32), loops])
  dst = jnp.concatenate([edge_index[1].astype(jnp.int32), loops])
  etot = src.shape[0]
  chunks = -(-etot // (NW * K))
  epad = chunks * NW * K
  src = jnp.concatenate([src, jnp.zeros((epad - etot,), jnp.int32)])
  dst = jnp.concatenate([dst, jnp.full((epad - etot,), DUMMY, jnp.int32)])

  pair = 2 * chunks
  c0n = max(1, round(pair * 190.0 / (245.0 + 190.0)))
  c1n = pair - c0n
  cmax = max(c0n, c1n)

  def _split(flat):
    rows = flat.reshape(NW * chunks, K)
    a = rows[:NS * c0n].reshape(NS, c0n, K)
    a = jnp.concatenate(
        [a, jnp.zeros((NS, cmax - c0n, K), jnp.int32)], axis=1)
    b = rows[NS * c0n:].reshape(NS, c1n, K)
    b = jnp.concatenate(
        [b, jnp.zeros((NS, cmax - c1n, K), jnp.int32)], axis=1)
    return jnp.concatenate([a, b], axis=0)

  srcr = _split(src)
  dstr = _split(dst)

  zeros_rows = jnp.zeros((128, D), jnp.float32)
  batch_pad = jnp.pad(batch.astype(jnp.int32), (0, NPAD - N),
                      constant_values=G).reshape(NPAD, 1)

  prop = _make_prop(c0n, c1n)
  deg = _make_deg(chunks)(dst.reshape(NW, chunks * K),
                          jnp.zeros((NPAD // 16, 16), jnp.float32))

  c0, dinv_b = _scale_x(x, deg)
  s1 = prop(c0, srcr, dstr, zeros_rows)
  c1 = _layer(s1, dinv_b, W1, b1.reshape(1, D))
  s2 = prop(c1, srcr, dstr, zeros_rows)
  c2 = _layer(s2, dinv_b, W2, b2.reshape(1, D))
  s3 = prop(c2, srcr, dstr, zeros_rows)
  return _pool(s3, dinv_b, batch_pad, W3, b3.reshape(1, D))


# speedup vs baseline: 1.0714x; 1.0714x over previous
import functools

import jax
import jax.numpy as jnp
from jax import lax
from jax.experimental import pallas as pl
from jax.experimental.pallas import tpu as pltpu
from jax.experimental.pallas import tpu_sc as plsc

N = 10000
G = 64
D = 128
NC = 2
NS = 16
NW = NC * NS
K = 128
NPAD = 10240
DUMMY = 10016
ROWS_PER_TILE = NPAD // NS
RB = 256
NBLK = NPAD // RB


@functools.lru_cache(maxsize=None)
def _make_prop(chunks: int):
  mesh = plsc.VectorSubcoreMesh(core_axis_name="c", subcore_axis_name="s")

  @functools.partial(
      pl.kernel,
      out_type=jax.ShapeDtypeStruct((NC, NPAD, D), jnp.float32),
      mesh=mesh,
      scratch_types=[
          pltpu.VMEM((chunks, K), jnp.int32),
          pltpu.VMEM((chunks, K), jnp.int32),
          pltpu.VMEM((K, D), jnp.float32),
          pltpu.VMEM_SHARED((NPAD, D), jnp.float32),
          pltpu.SemaphoreType.DMA,
          pltpu.SemaphoreType.DMA,
      ],
  )
  def prop(c_hbm, src_hbm, dst_hbm, zeros_hbm, out_hbm,
           src_v, dst_v, buf, acc, sem, sem_pre):
    cid = lax.axis_index("c")
    sid = lax.axis_index("s")
    wid = cid * NS + sid
    base = sid * ROWS_PER_TILE
    pltpu.async_copy(src_hbm.at[wid], src_v, sem_pre)
    pltpu.async_copy(dst_hbm.at[wid], dst_v, sem_pre)
    for z in range(ROWS_PER_TILE // 128):
      pltpu.async_copy(zeros_hbm, acc.at[pl.ds(base + z * 128, 128)], sem_pre)
    pltpu.make_async_copy(src_hbm.at[wid], src_v, sem_pre).wait()
    pltpu.make_async_copy(dst_hbm.at[wid], dst_v, sem_pre).wait()
    for z in range(ROWS_PER_TILE // 128):
      pltpu.make_async_copy(zeros_hbm,
                            acc.at[pl.ds(base + z * 128, 128)], sem_pre).wait()
    plsc.subcore_barrier()

    def body(j, carry):
      pltpu.async_copy(c_hbm.at[src_v.at[j]], buf, sem).wait()
      pltpu.sync_copy(buf, acc.at[dst_v.at[j]], add=True)
      return carry

    lax.fori_loop(0, chunks, body, 0)
    plsc.subcore_barrier()
    pltpu.sync_copy(acc.at[pl.ds(base, ROWS_PER_TILE)],
                    out_hbm.at[cid, pl.ds(base, ROWS_PER_TILE)])

  return prop


@functools.lru_cache(maxsize=None)
def _make_deg(chunks: int):
  mesh = plsc.VectorSubcoreMesh(core_axis_name="c", subcore_axis_name="s")

  @functools.partial(
      pl.kernel,
      out_type=jax.ShapeDtypeStruct((NW, NPAD // 16, 16), jnp.float32),
      mesh=mesh,
      scratch_types=[
          pltpu.VMEM((chunks * K,), jnp.int32),
          pltpu.VMEM((NPAD // 16, 16), jnp.float32),
      ],
      compiler_params=pltpu.CompilerParams(needs_layout_passes=False),
  )
  def deg(dst_hbm, zeros_hbm, out_hbm, dst_v, hist):
    cid = lax.axis_index("c")
    sid = lax.axis_index("s")
    wid = cid * NS + sid
    pltpu.sync_copy(dst_hbm.at[wid], dst_v)
    pltpu.sync_copy(zeros_hbm, hist)
    ones16 = jnp.ones((16,), jnp.float32)

    def body(i, carry):
      idx = dst_v[pl.ds(i * 16, 16)]
      plsc.addupdate_scatter(hist, [idx >> 4, idx & 15], ones16)
      return carry

    lax.fori_loop(0, (chunks * K) // 16, body, 0)
    pltpu.sync_copy(hist, out_hbm.at[wid])

  return deg


def _dinv_block(deg_blk):
  d = jnp.sum(deg_blk, axis=0)
  dinv16 = jnp.where(d > 0.0, lax.rsqrt(jnp.maximum(d, 1.0)), 0.0)
  sub = lax.broadcasted_iota(jnp.int32, (RB, 16), 0) // 16
  sel = (sub == lax.broadcasted_iota(jnp.int32, (RB, 16), 1)
         ).astype(jnp.float32)
  a = lax.dot_general(sel, dinv16, (((1,), (0,)), ((), ())),
                      preferred_element_type=jnp.float32)
  lane = lax.broadcasted_iota(jnp.int32, (RB, D), 0) % 16
  out = jnp.zeros((RB, D), jnp.float32)
  for b in range(16):
    out = out + jnp.where(lane == b, a[:, b:b + 1], 0.0)
  return out


def _scale_x_body(x_ref, deg_ref, c_ref, dinv_ref):
  dinv = _dinv_block(deg_ref[...])
  dinv_ref[...] = dinv
  c_ref[...] = x_ref[...] * dinv


def _scale_x(x, deg):
  return pl.pallas_call(
      _scale_x_body,
      grid=(NBLK,),
      in_specs=[
          pl.BlockSpec((RB, D), lambda i: (i, 0)),
          pl.BlockSpec((NW, RB // 16, 16), lambda i: (0, i, 0)),
      ],
      out_specs=[
          pl.BlockSpec((RB, D), lambda i: (i, 0)),
          pl.BlockSpec((RB, D), lambda i: (i, 0)),
      ],
      out_shape=[
          jax.ShapeDtypeStruct((NPAD, D), jnp.float32),
          jax.ShapeDtypeStruct((NPAD, D), jnp.float32),
      ],
      compiler_params=pltpu.CompilerParams(skip_device_barrier=True),
  )(x, deg)


def _layer_body(s_ref, dinv_ref, w_ref, b_ref, o_ref):
  dinv = dinv_ref[...]
  sp = (s_ref[0] + s_ref[1]) * dinv
  h = lax.dot_general(sp, w_ref[...], (((1,), (1,)), ((), ())),
                      preferred_element_type=jnp.float32)
  o_ref[...] = jnp.maximum(h + b_ref[...], 0.0) * dinv


def _layer(s, dinv_b, W, b):
  return pl.pallas_call(
      _layer_body,
      grid=(NBLK,),
      in_specs=[
          pl.BlockSpec((NC, RB, D), lambda i: (0, i, 0)),
          pl.BlockSpec((RB, D), lambda i: (i, 0)),
          pl.BlockSpec((D, D), lambda i: (0, 0)),
          pl.BlockSpec((1, D), lambda i: (0, 0)),
      ],
      out_specs=pl.BlockSpec((RB, D), lambda i: (i, 0)),
      out_shape=jax.ShapeDtypeStruct((NPAD, D), jnp.float32),
      compiler_params=pltpu.CompilerParams(skip_device_barrier=True),
  )(s, dinv_b, W, b)


def _pool_body(s_ref, dinv_ref, batch_ref, w_ref, b_ref, o_ref, s_acc, c_acc):
  i = pl.program_id(0)

  @pl.when(i == 0)
  def _():
    s_acc[...] = jnp.zeros_like(s_acc)
    c_acc[...] = jnp.zeros_like(c_acc)

  p = (s_ref[0] + s_ref[1]) * dinv_ref[...]
  bid = batch_ref[...]
  onehot = (lax.broadcasted_iota(jnp.int32, (RB, G), 1) == bid
            ).astype(jnp.float32)
  s_acc[...] += lax.dot_general(onehot, p, (((0,), (0,)), ((), ())),
                                preferred_element_type=jnp.float32)
  c_acc[...] += lax.dot_general(onehot, jnp.ones((RB, D), jnp.float32),
                                (((0,), (0,)), ((), ())),
                                preferred_element_type=jnp.float32)

  @pl.when(i == NBLK - 1)
  def _():
    cnt = c_acc[...]
    pooled = s_acc[...] / jnp.maximum(cnt, 1.0)
    res = lax.dot_general(pooled, w_ref[...], (((1,), (1,)), ((), ())),
                          preferred_element_type=jnp.float32)
    o_ref[...] = res + jnp.where(cnt > 0.0, b_ref[...], 0.0)


def _pool(s, dinv_b, batch_pad, W, b):
  return pl.pallas_call(
      _pool_body,
      grid=(NBLK,),
      in_specs=[
          pl.BlockSpec((NC, RB, D), lambda i: (0, i, 0)),
          pl.BlockSpec((RB, D), lambda i: (i, 0)),
          pl.BlockSpec((RB, 1), lambda i: (i, 0)),
          pl.BlockSpec((D, D), lambda i: (0, 0)),
          pl.BlockSpec((1, D), lambda i: (0, 0)),
      ],
      out_specs=pl.BlockSpec((G, D), lambda i: (0, 0)),
      out_shape=jax.ShapeDtypeStruct((G, D), jnp.float32),
      scratch_shapes=[
          pltpu.VMEM((G, D), jnp.float32),
          pltpu.VMEM((G, D), jnp.float32),
      ],
      compiler_params=pltpu.CompilerParams(skip_device_barrier=True),
  )(s, dinv_b, batch_pad, W, b)


def kernel(x, edge_index, batch, W1, b1, W2, b2, W3, b3):
  loops = jnp.arange(N, dtype=jnp.int32)
  src = jnp.concatenate([edge_index[0].astype(jnp.int32), loops])
  dst = jnp.concatenate([edge_index[1].astype(jnp.int32), loops])
  etot = src.shape[0]
  chunks = -(-etot // (NW * K))
  epad = chunks * NW * K
  src = jnp.concatenate([src, jnp.zeros((epad - etot,), jnp.int32)])
  dst = jnp.concatenate([dst, jnp.full((epad - etot,), DUMMY, jnp.int32)])
  srcr = src.reshape(NW, chunks, K)
  dstr = dst.reshape(NW, chunks, K)

  zeros_rows = jnp.zeros((128, D), jnp.float32)
  batch_pad = jnp.pad(batch.astype(jnp.int32), (0, NPAD - N),
                      constant_values=G).reshape(NPAD, 1)

  prop = _make_prop(chunks)
  deg = _make_deg(chunks)(dst.reshape(NW, chunks * K),
                          jnp.zeros((NPAD // 16, 16), jnp.float32))

  c0, dinv_b = _scale_x(x, deg)
  s1 = prop(c0, srcr, dstr, zeros_rows)
  c1 = _layer(s1, dinv_b, W1, b1.reshape(1, D))
  s2 = prop(c1, srcr, dstr, zeros_rows)
  c2 = _layer(s2, dinv_b, W2, b2.reshape(1, D))
  s3 = prop(c2, srcr, dstr, zeros_rows)
  return _pool(s3, dinv_b, batch_pad, W3, b3.reshape(1, D))
